# baseline (speedup 1.0000x reference)
import functools
import math

import jax
import jax.numpy as jnp
from jax import lax
from jax.experimental import pallas as pl
from jax.experimental.pallas import tpu as pltpu
from jax.experimental.pallas import tpu_sc as plsc

D_MODEL = 64
SCALE = math.sqrt(float(D_MODEL))
LANES = 16

_NC = 2
_NS = 16
_NW = _NC * _NS

_IDX_W = 128
_CHUNK = 512
_GPC = _CHUNK // _IDX_W


def _body(x_hbm, table_hbm, out_hbm, idx_v, rows_v, sem):
    wid = lax.axis_index("s") * _NC + lax.axis_index("c")
    n_rows = out_hbm.shape[0]
    b_per_w = n_rows // _NW
    n_chunks = b_per_w // _CHUNK
    idx_rows_per_w = b_per_w // _IDX_W
    base_idx_row = wid * idx_rows_per_w
    base_row = wid * b_per_w

    def chunk(g, carry):
        pltpu.sync_copy(x_hbm.at[pl.ds(base_idx_row + g * _GPC, _GPC)], idx_v)
        handles = []
        for j in range(_GPC):
            handles.append(pltpu.async_copy(
                table_hbm.at[idx_v.at[j]],
                rows_v.at[pl.ds(j * _IDX_W, _IDX_W)],
                sem))
        for h in handles:
            h.wait()

        def row(r, c2):
            for c in range(D_MODEL // LANES):
                sl = pl.ds(c * LANES, LANES)
                rows_v[r, sl] = rows_v[r, sl] * SCALE
            return c2
        lax.fori_loop(0, _CHUNK, row, 0)

        pltpu.sync_copy(rows_v, out_hbm.at[pl.ds(base_row + g * _CHUNK, _CHUNK)])
        return carry

    lax.fori_loop(0, n_chunks, chunk, 0)


def kernel(x, table):
    S0, S1 = x.shape
    B = S0 * S1
    xf = x.reshape(B // _IDX_W, _IDX_W).astype(jnp.int32)

    fn = functools.partial(
        pl.kernel,
        out_type=jax.ShapeDtypeStruct((B, D_MODEL), jnp.float32),
        mesh=plsc.VectorSubcoreMesh(core_axis_name="c", subcore_axis_name="s"),
        scratch_types=[
            pltpu.VMEM((_GPC, _IDX_W), jnp.int32),
            pltpu.VMEM((_CHUNK, D_MODEL), jnp.float32),
            pltpu.SemaphoreType.DMA,
        ],
        compiler_params=pltpu.CompilerParams(use_tc_tiling_on_sc=False),
    )(_body)
    out = fn(xf, table)
    return out.reshape(S0, S1, D_MODEL)


# speedup vs baseline: 1.1391x; 1.1391x over previous
import functools
import math

import jax
import jax.numpy as jnp
from jax import lax
from jax.experimental import pallas as pl
from jax.experimental.pallas import tpu as pltpu
from jax.experimental.pallas import tpu_sc as plsc

D_MODEL = 64
SCALE = math.sqrt(float(D_MODEL))
LANES = 16

_NC = 2
_NS = 16
_NW = _NC * _NS

_IDX_W = 128
_CHUNK = 512
_GPC = _CHUNK // _IDX_W


def _body(x_hbm, table_hbm, out_hbm, idx_v, rows_v, sem_i, sem_g, sem_s):
    wid = lax.axis_index("s") * _NC + lax.axis_index("c")
    b_per_w = out_hbm.shape[0] // _NW
    n_chunks = b_per_w // _CHUNK
    base_idx_row = wid * (b_per_w // _IDX_W)
    base_row = wid * b_per_w

    def idx_rows(g):
        return x_hbm.at[pl.ds(base_idx_row + g * _GPC, _GPC)]

    def out_rows(g):
        return out_hbm.at[pl.ds(base_row + g * _CHUNK, _CHUNK)]

    def fire_gathers(ib, rb):
        for j in range(_GPC):
            pltpu.async_copy(table_hbm.at[idx_v.at[ib, j]],
                             rows_v.at[rb, pl.ds(j * _IDX_W, _IDX_W)],
                             sem_g)

    def wait_gathers(rb):
        pltpu.make_async_copy(table_hbm.at[pl.ds(0, _CHUNK)],
                              rows_v.at[rb], sem_g).wait()

    def wait_scatter(g, rb):
        pltpu.make_async_copy(rows_v.at[rb], out_rows(g), sem_s).wait()

    def scale(rb):
        @plsc.parallel_loop(0, _CHUNK, step=1, unroll=8)
        def _(r):
            for c in range(D_MODEL // LANES):
                sl = pl.ds(c * LANES, LANES)
                rows_v[rb, r, sl] = rows_v[rb, r, sl] * SCALE

    def do_chunk(g, p, first, has_next, has_idx2):
        q = 1 - p
        if has_next:
            pltpu.make_async_copy(idx_rows(g + 1), idx_v.at[q], sem_i).wait()
            if not first:
                wait_scatter(g - 1, q)
            fire_gathers(q, q)
        elif not first:
            wait_scatter(g - 1, q)
        wait_gathers(p)
        if has_idx2:
            pltpu.async_copy(idx_rows(g + 2), idx_v.at[p], sem_i)
        scale(p)
        pltpu.async_copy(rows_v.at[p], out_rows(g), sem_s)

    pltpu.sync_copy(idx_rows(0), idx_v.at[0])
    fire_gathers(0, 0)
    pltpu.async_copy(idx_rows(1), idx_v.at[1], sem_i)

    do_chunk(0, 0, True, True, True)
    do_chunk(1, 1, False, True, True)

    def pair(k, carry):
        g = 2 * k
        do_chunk(g, 0, False, True, True)
        do_chunk(g + 1, 1, False, True, True)
        return carry
    lax.fori_loop(1, n_chunks // 2 - 1, pair, 0)

    do_chunk(n_chunks - 2, 0, False, True, False)
    do_chunk(n_chunks - 1, 1, False, False, False)
    wait_scatter(n_chunks - 1, 1)


def kernel(x, table):
    S0, S1 = x.shape
    B = S0 * S1
    xf = x.reshape(B // _IDX_W, _IDX_W).astype(jnp.int32)

    fn = functools.partial(
        pl.kernel,
        out_type=jax.ShapeDtypeStruct((B, D_MODEL), jnp.float32),
        mesh=plsc.VectorSubcoreMesh(core_axis_name="c", subcore_axis_name="s"),
        scratch_types=[
            pltpu.VMEM((2, _GPC, _IDX_W), jnp.int32),
            pltpu.VMEM((2, _CHUNK, D_MODEL), jnp.float32),
            pltpu.SemaphoreType.DMA,
            pltpu.SemaphoreType.DMA,
            pltpu.SemaphoreType.DMA,
        ],
        compiler_params=pltpu.CompilerParams(use_tc_tiling_on_sc=False),
    )(_body)
    out = fn(xf, table)
    return out.reshape(S0, S1, D_MODEL)
